# Initial kernel scaffold; baseline (speedup 1.0000x reference)
import functools

import jax
import jax.numpy as jnp
from jax.experimental import pallas as pl
from jax.experimental.pallas import tpu as pltpu

T = 2048
D = 1024
F = 512
E = 8
RSF = 2.5
EPS = 1e-6

TBLK = 128


def _moe_body(x_ref, rw_ref, bias_ref, wg_ref, wu_ref, wd_ref,
              swg_ref, swu_ref, swd_ref, ln_ref, out_ref):
    x = x_ref[...]
    var = jnp.mean(x * x, axis=-1, keepdims=True)
    h = x * jax.lax.rsqrt(var + EPS) * ln_ref[...]
    hb = h.astype(jnp.bfloat16)

    logits = jnp.dot(h, rw_ref[...], preferred_element_type=jnp.float32,
                     precision=jax.lax.Precision.HIGHEST)
    scores = jax.nn.sigmoid(logits)
    sfc = scores + bias_ref[...]

    eidx = jax.lax.broadcasted_iota(jnp.int32, (TBLK, E), 1)
    neg = jnp.float32(-jnp.inf)
    m1 = jnp.max(sfc, axis=1, keepdims=True)
    i1 = jnp.min(jnp.where(sfc == m1, eidx, E), axis=1, keepdims=True)
    sfc2 = jnp.where(eidx == i1, neg, sfc)
    m2 = jnp.max(sfc2, axis=1, keepdims=True)
    i2 = jnp.min(jnp.where(sfc2 == m2, eidx, E), axis=1, keepdims=True)

    w1 = jnp.sum(jnp.where(eidx == i1, scores, 0.0), axis=1, keepdims=True)
    w2 = jnp.sum(jnp.where(eidx == i2, scores, 0.0), axis=1, keepdims=True)
    denom = w1 + w2 + 1e-20
    combine = (jnp.where(eidx == i1, w1, 0.0)
               + jnp.where(eidx == i2, w2, 0.0)) / denom * RSF

    sg = jnp.dot(hb, swg_ref[...], preferred_element_type=jnp.float32)
    su = jnp.dot(hb, swu_ref[...], preferred_element_type=jnp.float32)
    sinter = (jax.nn.silu(sg) * su).astype(jnp.bfloat16)
    acc = jnp.dot(sinter, swd_ref[...], preferred_element_type=jnp.float32)

    for e in range(E):
        a1 = jnp.dot(hb, wg_ref[e], preferred_element_type=jnp.float32)
        a2 = jnp.dot(hb, wu_ref[e], preferred_element_type=jnp.float32)
        inter = (jax.nn.silu(a1) * a2).astype(jnp.bfloat16)
        ye = jnp.dot(inter, wd_ref[e], preferred_element_type=jnp.float32)
        acc = acc + ye * combine[:, e:e + 1]

    out_ref[...] = acc


@jax.jit
def kernel(hidden_states, router_w, expert_bias, w_gate, w_up, w_down,
           sw_gate, sw_up, sw_down, ln_w):
    bf = jnp.bfloat16
    grid = (T // TBLK,)
    full = lambda *s: pl.BlockSpec(s, lambda i: (0,) * len(s))
    out = pl.pallas_call(
        _moe_body,
        grid=grid,
        in_specs=[
            pl.BlockSpec((TBLK, D), lambda i: (i, 0)),
            full(D, E),
            full(1, E),
            full(E, D, F),
            full(E, D, F),
            full(E, F, D),
            full(D, F),
            full(D, F),
            full(F, D),
            full(1, D),
        ],
        out_specs=pl.BlockSpec((TBLK, D), lambda i: (i, 0)),
        out_shape=jax.ShapeDtypeStruct((T, D), jnp.float32),
    )(hidden_states, router_w, expert_bias.reshape(1, E),
      w_gate.astype(bf), w_up.astype(bf), w_down.astype(bf),
      sw_gate.astype(bf), sw_up.astype(bf), sw_down.astype(bf),
      ln_w.reshape(1, D))
    return out


# speedup vs baseline: 1.7197x; 1.7197x over previous
import functools

import jax
import jax.numpy as jnp
from jax.experimental import pallas as pl
from jax.experimental.pallas import tpu as pltpu

T = 2048
D = 1024
F = 512
E = 8
RSF = 2.5
EPS = 1e-6

TBLK = 128


def _moe_body(x_ref, rw_ref, bias_ref, wg_ref, wu_ref, wd_ref,
              swg_ref, swu_ref, swd_ref, ln_ref, out_ref):
    x = x_ref[...]
    var = jnp.mean(x * x, axis=-1, keepdims=True)
    h = x * jax.lax.rsqrt(var + EPS) * ln_ref[...]
    hb = h.astype(jnp.bfloat16)

    logits = jnp.dot(h, rw_ref[...], preferred_element_type=jnp.float32)
    scores = jax.nn.sigmoid(logits)
    sfc = scores + bias_ref[...]

    eidx = jax.lax.broadcasted_iota(jnp.int32, (TBLK, E), 1)
    neg = jnp.float32(-jnp.inf)
    m1 = jnp.max(sfc, axis=1, keepdims=True)
    i1 = jnp.min(jnp.where(sfc == m1, eidx, E), axis=1, keepdims=True)
    sfc2 = jnp.where(eidx == i1, neg, sfc)
    m2 = jnp.max(sfc2, axis=1, keepdims=True)
    i2 = jnp.min(jnp.where(sfc2 == m2, eidx, E), axis=1, keepdims=True)

    w1 = jnp.sum(jnp.where(eidx == i1, scores, 0.0), axis=1, keepdims=True)
    w2 = jnp.sum(jnp.where(eidx == i2, scores, 0.0), axis=1, keepdims=True)
    denom = w1 + w2 + 1e-20
    combine = (jnp.where(eidx == i1, w1, 0.0)
               + jnp.where(eidx == i2, w2, 0.0)) / denom * RSF

    sg = jnp.dot(hb, swg_ref[...], preferred_element_type=jnp.float32)
    su = jnp.dot(hb, swu_ref[...], preferred_element_type=jnp.float32)
    sinter = (jax.nn.silu(sg) * su).astype(jnp.bfloat16)
    acc = jnp.dot(sinter, swd_ref[...], preferred_element_type=jnp.float32)

    for e in range(E):
        a1 = jnp.dot(hb, wg_ref[e], preferred_element_type=jnp.float32)
        a2 = jnp.dot(hb, wu_ref[e], preferred_element_type=jnp.float32)
        inter = (jax.nn.silu(a1) * a2).astype(jnp.bfloat16)
        ye = jnp.dot(inter, wd_ref[e], preferred_element_type=jnp.float32)
        acc = acc + ye * combine[:, e:e + 1]

    out_ref[...] = acc


@jax.jit
def kernel(hidden_states, router_w, expert_bias, w_gate, w_up, w_down,
           sw_gate, sw_up, sw_down, ln_w):
    bf = jnp.bfloat16
    grid = (T // TBLK,)
    full = lambda *s: pl.BlockSpec(s, lambda i: (0,) * len(s))
    out = pl.pallas_call(
        _moe_body,
        grid=grid,
        in_specs=[
            pl.BlockSpec((TBLK, D), lambda i: (i, 0)),
            full(D, E),
            full(1, E),
            full(E, D, F),
            full(E, D, F),
            full(E, F, D),
            full(D, F),
            full(D, F),
            full(F, D),
            full(1, D),
        ],
        out_specs=pl.BlockSpec((TBLK, D), lambda i: (i, 0)),
        out_shape=jax.ShapeDtypeStruct((T, D), jnp.float32),
    )(hidden_states, router_w, expert_bias.reshape(1, E),
      w_gate.astype(bf), w_up.astype(bf), w_down.astype(bf),
      sw_gate.astype(bf), sw_up.astype(bf), sw_down.astype(bf),
      ln_w.reshape(1, D))
    return out


# speedup vs baseline: 2.0561x; 1.1956x over previous
import functools

import jax
import jax.numpy as jnp
from jax.experimental import pallas as pl
from jax.experimental.pallas import tpu as pltpu

T = 2048
D = 1024
F = 512
E = 8
RSF = 2.5
EPS = 1e-6

TBLK = 128


def _moe_body(x_ref, rw_ref, bias_ref, wg_ref, wu_ref, wd_ref,
              swg_ref, swu_ref, swd_ref, ln_ref, out_ref):
    x = x_ref[...]
    var = jnp.mean(x * x, axis=-1, keepdims=True)
    h = x * jax.lax.rsqrt(var + EPS) * ln_ref[...]
    hb = h

    logits = jnp.dot(h, rw_ref[...], preferred_element_type=jnp.float32)
    scores = jax.nn.sigmoid(logits)
    sfc = scores + bias_ref[...]

    eidx = jax.lax.broadcasted_iota(jnp.int32, (TBLK, E), 1)
    neg = jnp.float32(-jnp.inf)
    m1 = jnp.max(sfc, axis=1, keepdims=True)
    i1 = jnp.min(jnp.where(sfc == m1, eidx, E), axis=1, keepdims=True)
    sfc2 = jnp.where(eidx == i1, neg, sfc)
    m2 = jnp.max(sfc2, axis=1, keepdims=True)
    i2 = jnp.min(jnp.where(sfc2 == m2, eidx, E), axis=1, keepdims=True)

    w1 = jnp.sum(jnp.where(eidx == i1, scores, 0.0), axis=1, keepdims=True)
    w2 = jnp.sum(jnp.where(eidx == i2, scores, 0.0), axis=1, keepdims=True)
    denom = w1 + w2 + 1e-20
    combine = (jnp.where(eidx == i1, w1, 0.0)
               + jnp.where(eidx == i2, w2, 0.0)) / denom * RSF

    sg = jnp.dot(hb, swg_ref[...], preferred_element_type=jnp.float32)
    su = jnp.dot(hb, swu_ref[...], preferred_element_type=jnp.float32)
    sinter = (jax.nn.silu(sg) * su).astype(jnp.bfloat16)
    acc = jnp.dot(sinter, swd_ref[...], preferred_element_type=jnp.float32)

    for e in range(E):
        a1 = jnp.dot(hb, wg_ref[e], preferred_element_type=jnp.float32)
        a2 = jnp.dot(hb, wu_ref[e], preferred_element_type=jnp.float32)
        inter = jax.nn.silu(a1) * a2
        ye = jnp.dot(inter, wd_ref[e], preferred_element_type=jnp.float32)
        acc = acc + ye * combine[:, e:e + 1]

    out_ref[...] = acc


@jax.jit
def kernel(hidden_states, router_w, expert_bias, w_gate, w_up, w_down,
           sw_gate, sw_up, sw_down, ln_w):
    bf = jnp.bfloat16
    grid = (T // TBLK,)
    full = lambda *s: pl.BlockSpec(s, lambda i: (0,) * len(s))
    out = pl.pallas_call(
        _moe_body,
        grid=grid,
        in_specs=[
            pl.BlockSpec((TBLK, D), lambda i: (i, 0)),
            full(D, E),
            full(1, E),
            full(E, D, F),
            full(E, D, F),
            full(E, F, D),
            full(D, F),
            full(D, F),
            full(F, D),
            full(1, D),
        ],
        out_specs=pl.BlockSpec((TBLK, D), lambda i: (i, 0)),
        out_shape=jax.ShapeDtypeStruct((T, D), jnp.float32),
    )(hidden_states, router_w, expert_bias.reshape(1, E),
      w_gate, w_up, w_down, sw_gate, sw_up, sw_down,
      ln_w.reshape(1, D))
    return out
